# baseline (speedup 1.0000x reference)
import functools

import jax
import jax.numpy as jnp
from jax import lax
from jax.experimental import pallas as pl
from jax.experimental.pallas import tpu as pltpu
from jax.experimental.pallas import tpu_sc as plsc

N = 50000
NT = 51200
E = 800000
NC = 2
NS = 16
NW = NC * NS
B = 128
K = 196
EP = NW * K * B
CHK = 14
RPT = NT // NS
DEGW = 16
IN_DIM = 32
OUT_DIM = 128

_MESH = plsc.VectorSubcoreMesh(core_axis_name="c", subcore_axis_name="s")
_SC_PARAMS = pltpu.CompilerParams(use_tc_tiling_on_sc=False,
                                  needs_layout_passes=False)


def _zero_vmem(ref, nrow, ncol):
    z16 = jnp.zeros((16,), jnp.float32)

    def body(i, carry):
        for j in range(ncol // 16):
            ref[i, pl.ds(j * 16, 16)] = z16
        return carry

    lax.fori_loop(0, nrow, body, 0)


def _zero_table(tab, zer, s):
    def body(r, carry):
        pltpu.sync_copy(zer, tab.at[pl.ds(s * RPT + r * B, B)])
        return carry

    lax.fori_loop(0, RPT // B, body, 0)


def _deg_body(dst_hbm, out_hbm, tab, idx_v, val_v, zer_v):
    c = lax.axis_index("c")
    s = lax.axis_index("s")
    wid = c * NS + s
    one16 = jnp.ones((16,), jnp.float32)

    def fill(i, carry):
        val_v[i, :] = one16
        zer_v[i, :] = jnp.zeros((16,), jnp.float32)
        return carry

    lax.fori_loop(0, B, fill, 0)
    _zero_table(tab, zer_v, s)
    plsc.subcore_barrier()

    pltpu.sync_copy(dst_hbm.at[wid], idx_v)

    def body(g, carry):
        pltpu.sync_copy(val_v, tab.at[idx_v.at[g]], add=True)
        return carry

    lax.fori_loop(0, K, body, 0)
    plsc.subcore_barrier()
    pltpu.sync_copy(tab.at[pl.ds(s * RPT, RPT)], out_hbm.at[c, pl.ds(s * RPT, RPT)])


def _make_deg(interpret=False):
    return pl.kernel(
        _deg_body,
        out_type=jax.ShapeDtypeStruct((NC, NT, DEGW), jnp.float32),
        mesh=_MESH,
        scratch_types=[
            pltpu.VMEM_SHARED((NT, DEGW), jnp.float32),
            pltpu.VMEM((K, B), jnp.int32),
            pltpu.VMEM((B, DEGW), jnp.float32),
            pltpu.VMEM((B, DEGW), jnp.float32),
        ],
        interpret=interpret,
        compiler_params=_SC_PARAMS,
    )


def _scat_body(src_hbm, dst_hbm, xs_hbm, out_hbm, tab, sidx, didx, rows, zer, sem):
    c = lax.axis_index("c")
    s = lax.axis_index("s")
    wid = c * NS + s

    _zero_vmem(zer, B, IN_DIM)
    _zero_table(tab, zer, s)
    plsc.subcore_barrier()

    def outer(o, carry):
        pltpu.sync_copy(src_hbm.at[wid, pl.ds(o * CHK, CHK)], sidx)
        pltpu.sync_copy(dst_hbm.at[wid, pl.ds(o * CHK, CHK)], didx)

        def body(g, carry2):
            pltpu.async_copy(xs_hbm.at[sidx.at[g]], rows, sem).wait()
            pltpu.sync_copy(rows, tab.at[didx.at[g]], add=True)
            return carry2

        lax.fori_loop(0, CHK, body, 0)
        return carry

    lax.fori_loop(0, K // CHK, outer, 0)
    plsc.subcore_barrier()
    pltpu.sync_copy(tab.at[pl.ds(s * RPT, RPT)], out_hbm.at[c, pl.ds(s * RPT, RPT)])


def _make_scat(interpret=False):
    return pl.kernel(
        _scat_body,
        out_type=jax.ShapeDtypeStruct((NC, NT, IN_DIM), jnp.float32),
        mesh=_MESH,
        scratch_types=[
            pltpu.VMEM_SHARED((NT, IN_DIM), jnp.float32),
            pltpu.VMEM((CHK, B), jnp.int32),
            pltpu.VMEM((CHK, B), jnp.int32),
            pltpu.VMEM((B, IN_DIM), jnp.float32),
            pltpu.VMEM((B, IN_DIM), jnp.float32),
            pltpu.SemaphoreType.DMA,
        ],
        interpret=interpret,
        compiler_params=_SC_PARAMS,
    )


def _dec_body(src_hbm, dst_hbm, z_hbm, out_hbm, sidx, didx,
              zs0, zd0, zs1, zd1, res, sa0, sb0, sa1, sb1):
    c = lax.axis_index("c")
    s = lax.axis_index("s")
    wid = c * NS + s
    lane = jnp.arange(16, dtype=jnp.int32)

    pltpu.sync_copy(src_hbm.at[wid], sidx)
    pltpu.sync_copy(dst_hbm.at[wid], didx)

    def issue(g, zsb, zdb, sa, sb):
        gg = jnp.minimum(g, K - 1)
        pltpu.async_copy(z_hbm.at[sidx.at[gg]], zsb, sa)
        pltpu.async_copy(z_hbm.at[didx.at[gg]], zdb, sb)

    def wait(zsb, zdb, sa, sb):
        pltpu.make_async_copy(z_hbm.at[sidx.at[0]], zsb, sa).wait()
        pltpu.make_async_copy(z_hbm.at[didx.at[0]], zdb, sb).wait()

    def compute(zsb, zdb, g):
        def grp(t, carry2):
            rows = t * 16 + lane
            accs = [jnp.zeros((16,), jnp.float32) for _ in range(4)]
            for j in range(OUT_DIM):
                col = jnp.full((16,), j, jnp.int32)
                a = plsc.load_gather(zsb, [rows, col])
                bb = plsc.load_gather(zdb, [rows, col])
                accs[j % 4] = accs[j % 4] + a * bb
            acc = (accs[0] + accs[1]) + (accs[2] + accs[3])
            res[pl.ds(t * 16, 16)] = 1.0 / (1.0 + jnp.exp(-acc)) + 1e-15
            return carry2

        lax.fori_loop(0, B // 16, grp, 0)
        pltpu.sync_copy(res, out_hbm.at[wid, g])

    issue(0, zs0, zd0, sa0, sb0)

    def outer(o, carry):
        g0 = 2 * o
        issue(g0 + 1, zs1, zd1, sa1, sb1)
        wait(zs0, zd0, sa0, sb0)
        compute(zs0, zd0, g0)
        issue(g0 + 2, zs0, zd0, sa0, sb0)
        wait(zs1, zd1, sa1, sb1)
        compute(zs1, zd1, g0 + 1)
        return carry

    lax.fori_loop(0, K // 2, outer, 0)
    wait(zs0, zd0, sa0, sb0)


def _make_dec(interpret=False):
    return pl.kernel(
        _dec_body,
        out_type=jax.ShapeDtypeStruct((NW, K, B), jnp.float32),
        mesh=_MESH,
        scratch_types=[
            pltpu.VMEM((K, B), jnp.int32),
            pltpu.VMEM((K, B), jnp.int32),
            pltpu.VMEM((B, OUT_DIM), jnp.float32),
            pltpu.VMEM((B, OUT_DIM), jnp.float32),
            pltpu.VMEM((B, OUT_DIM), jnp.float32),
            pltpu.VMEM((B, OUT_DIM), jnp.float32),
            pltpu.VMEM((B,), jnp.float32),
            pltpu.SemaphoreType.DMA,
            pltpu.SemaphoreType.DMA,
            pltpu.SemaphoreType.DMA,
            pltpu.SemaphoreType.DMA,
        ],
        interpret=interpret,
        compiler_params=_SC_PARAMS,
    )


def _dense_body(t_ref, xs_ref, d_ref, w_ref, b_ref, z_ref):
    t = t_ref[0] + t_ref[1] + xs_ref[...]
    y = jnp.dot(t, w_ref[...], preferred_element_type=jnp.float32)
    z_ref[...] = jnp.maximum(y * d_ref[...] + b_ref[...], 0.0)


def _make_dense(interpret=False):
    blk = 1600
    return pl.pallas_call(
        _dense_body,
        grid=(NT // blk,),
        in_specs=[
            pl.BlockSpec((NC, blk, IN_DIM), lambda i: (0, i, 0)),
            pl.BlockSpec((blk, IN_DIM), lambda i: (i, 0)),
            pl.BlockSpec((blk, 1), lambda i: (i, 0)),
            pl.BlockSpec((IN_DIM, OUT_DIM), lambda i: (0, 0)),
            pl.BlockSpec((1, OUT_DIM), lambda i: (0, 0)),
        ],
        out_specs=pl.BlockSpec((blk, OUT_DIM), lambda i: (i, 0)),
        out_shape=jax.ShapeDtypeStruct((NT, OUT_DIM), jnp.float32),
        interpret=interpret,
    )


def _build(interpret=False):
    return (_make_deg(interpret), _make_scat(interpret), _make_dec(interpret),
            _make_dense(interpret))


def kernel(x, edge_index, W, b):
    deg_call, scat_call, dec_call, dense_call = _build()

    src = edge_index[0].astype(jnp.int32)
    dst = edge_index[1].astype(jnp.int32)
    pad = EP - E
    srcp = jnp.concatenate([src, jnp.zeros((pad,), jnp.int32)]).reshape(NW, K, B)
    dstp_s = jnp.concatenate([dst, jnp.full((pad,), N, jnp.int32)]).reshape(NW, K, B)
    dstp_d = jnp.concatenate([dst, jnp.zeros((pad,), jnp.int32)]).reshape(NW, K, B)

    degt = deg_call(dstp_s)
    deg = degt[0, :, 0] + degt[1, :, 0] + 1.0
    dinv = lax.rsqrt(deg)
    x_pad = jnp.concatenate([x, jnp.zeros((NT - N, IN_DIM), jnp.float32)])
    xs = x_pad * dinv[:, None]

    t_tab = scat_call(srcp, dstp_s, xs)
    z = dense_call(t_tab, xs, dinv.reshape(NT, 1), W, b.reshape(1, OUT_DIM))

    outr = dec_call(srcp, dstp_d, z)
    adj_pred = outr.reshape(EP)[:E]
    return (adj_pred, edge_index)


# speedup vs baseline: 3.3710x; 3.3710x over previous
import functools

import jax
import jax.numpy as jnp
from jax import lax
from jax.experimental import pallas as pl
from jax.experimental.pallas import tpu as pltpu
from jax.experimental.pallas import tpu_sc as plsc

N = 50000
NT = 51200
E = 800000
NC = 2
NS = 16
NW = NC * NS
B = 128
K = 196
EP = NW * K * B
CHK = 14
RPT = NT // NS
DEGW = 16
IN_DIM = 32
OUT_DIM = 128

_MESH = plsc.VectorSubcoreMesh(core_axis_name="c", subcore_axis_name="s")
_SC_PARAMS = pltpu.CompilerParams(use_tc_tiling_on_sc=False,
                                  needs_layout_passes=False)


def _zero_vmem(ref, nrow, ncol):
    z16 = jnp.zeros((16,), jnp.float32)

    def body(i, carry):
        for j in range(ncol // 16):
            ref[i, pl.ds(j * 16, 16)] = z16
        return carry

    lax.fori_loop(0, nrow, body, 0)


def _zero_table(tab, zer, s):
    def body(r, carry):
        pltpu.sync_copy(zer, tab.at[pl.ds(s * RPT + r * B, B)])
        return carry

    lax.fori_loop(0, RPT // B, body, 0)


def _deg_body(dst_hbm, out_hbm, tab, idx_v, val_v, zer_v):
    c = lax.axis_index("c")
    s = lax.axis_index("s")
    wid = c * NS + s
    one16 = jnp.ones((16,), jnp.float32)

    def fill(i, carry):
        val_v[i, :] = one16
        zer_v[i, :] = jnp.zeros((16,), jnp.float32)
        return carry

    lax.fori_loop(0, B, fill, 0)
    _zero_table(tab, zer_v, s)
    plsc.subcore_barrier()

    pltpu.sync_copy(dst_hbm.at[wid], idx_v)

    def body(g, carry):
        pltpu.sync_copy(val_v, tab.at[idx_v.at[g]], add=True)
        return carry

    lax.fori_loop(0, K, body, 0)
    plsc.subcore_barrier()
    pltpu.sync_copy(tab.at[pl.ds(s * RPT, RPT)], out_hbm.at[c, pl.ds(s * RPT, RPT)])


def _make_deg(interpret=False):
    return pl.kernel(
        _deg_body,
        out_type=jax.ShapeDtypeStruct((NC, NT, DEGW), jnp.float32),
        mesh=_MESH,
        scratch_types=[
            pltpu.VMEM_SHARED((NT, DEGW), jnp.float32),
            pltpu.VMEM((K, B), jnp.int32),
            pltpu.VMEM((B, DEGW), jnp.float32),
            pltpu.VMEM((B, DEGW), jnp.float32),
        ],
        interpret=interpret,
        compiler_params=_SC_PARAMS,
    )


def _scat_body(src_hbm, dst_hbm, xs_hbm, out_hbm, tab, sidx, didx, rows, zer, sem):
    c = lax.axis_index("c")
    s = lax.axis_index("s")
    wid = c * NS + s

    _zero_vmem(zer, B, IN_DIM)
    _zero_table(tab, zer, s)
    plsc.subcore_barrier()

    def outer(o, carry):
        pltpu.sync_copy(src_hbm.at[wid, pl.ds(o * CHK, CHK)], sidx)
        pltpu.sync_copy(dst_hbm.at[wid, pl.ds(o * CHK, CHK)], didx)

        def body(g, carry2):
            pltpu.async_copy(xs_hbm.at[sidx.at[g]], rows, sem).wait()
            pltpu.sync_copy(rows, tab.at[didx.at[g]], add=True)
            return carry2

        lax.fori_loop(0, CHK, body, 0)
        return carry

    lax.fori_loop(0, K // CHK, outer, 0)
    plsc.subcore_barrier()
    pltpu.sync_copy(tab.at[pl.ds(s * RPT, RPT)], out_hbm.at[c, pl.ds(s * RPT, RPT)])


def _make_scat(interpret=False):
    return pl.kernel(
        _scat_body,
        out_type=jax.ShapeDtypeStruct((NC, NT, IN_DIM), jnp.float32),
        mesh=_MESH,
        scratch_types=[
            pltpu.VMEM_SHARED((NT, IN_DIM), jnp.float32),
            pltpu.VMEM((CHK, B), jnp.int32),
            pltpu.VMEM((CHK, B), jnp.int32),
            pltpu.VMEM((B, IN_DIM), jnp.float32),
            pltpu.VMEM((B, IN_DIM), jnp.float32),
            pltpu.SemaphoreType.DMA,
        ],
        interpret=interpret,
        compiler_params=_SC_PARAMS,
    )


def _dec_body(src_hbm, dst_hbm, z_hbm, out_hbm, sidx, didx,
              zs0, zd0, zs1, zd1, res, stage, sa0, sb0, sa1, sb1):
    c = lax.axis_index("c")
    s = lax.axis_index("s")
    wid = c * NS + s
    lane = jnp.arange(16, dtype=jnp.int32)

    pltpu.sync_copy(src_hbm.at[wid], sidx)
    pltpu.sync_copy(dst_hbm.at[wid], didx)

    def issue(g, zsb, zdb, sa, sb):
        gg = jnp.minimum(g, K - 1)
        pltpu.async_copy(z_hbm.at[sidx.at[gg]], zsb, sa)
        pltpu.async_copy(z_hbm.at[didx.at[gg]], zdb, sb)

    def wait(zsb, zdb, sa, sb):
        pltpu.make_async_copy(z_hbm.at[sidx.at[0]], zsb, sa).wait()
        pltpu.make_async_copy(z_hbm.at[didx.at[0]], zdb, sb).wait()

    def compute(zsb, zdb, g):
        def grp(t, carry2):
            for k in range(16):
                e = t * 16 + k
                prods = [zsb[e, pl.ds(j * 16, 16)] * zdb[e, pl.ds(j * 16, 16)]
                         for j in range(OUT_DIM // 16)]
                while len(prods) > 1:
                    prods = [prods[i] + prods[i + 1]
                             for i in range(0, len(prods), 2)]
                stage[k, :] = prods[0]
            sums = [jnp.zeros((16,), jnp.float32) for _ in range(4)]
            for j in range(16):
                col = (lane + j) & 15
                sums[j % 4] = sums[j % 4] + plsc.load_gather(stage, [lane, col])
            acc = (sums[0] + sums[1]) + (sums[2] + sums[3])
            res[pl.ds(t * 16, 16)] = 1.0 / (1.0 + jnp.exp(-acc)) + 1e-15
            return carry2

        lax.fori_loop(0, B // 16, grp, 0)
        pltpu.sync_copy(res, out_hbm.at[wid, g])

    issue(0, zs0, zd0, sa0, sb0)

    def outer(o, carry):
        g0 = 2 * o
        issue(g0 + 1, zs1, zd1, sa1, sb1)
        wait(zs0, zd0, sa0, sb0)
        compute(zs0, zd0, g0)
        issue(g0 + 2, zs0, zd0, sa0, sb0)
        wait(zs1, zd1, sa1, sb1)
        compute(zs1, zd1, g0 + 1)
        return carry

    lax.fori_loop(0, K // 2, outer, 0)
    wait(zs0, zd0, sa0, sb0)


def _make_dec(interpret=False):
    return pl.kernel(
        _dec_body,
        out_type=jax.ShapeDtypeStruct((NW, K, B), jnp.float32),
        mesh=_MESH,
        scratch_types=[
            pltpu.VMEM((K, B), jnp.int32),
            pltpu.VMEM((K, B), jnp.int32),
            pltpu.VMEM((B, OUT_DIM), jnp.float32),
            pltpu.VMEM((B, OUT_DIM), jnp.float32),
            pltpu.VMEM((B, OUT_DIM), jnp.float32),
            pltpu.VMEM((B, OUT_DIM), jnp.float32),
            pltpu.VMEM((B,), jnp.float32),
            pltpu.VMEM((16, 16), jnp.float32),
            pltpu.SemaphoreType.DMA,
            pltpu.SemaphoreType.DMA,
            pltpu.SemaphoreType.DMA,
            pltpu.SemaphoreType.DMA,
        ],
        interpret=interpret,
        compiler_params=_SC_PARAMS,
    )


def _dense_body(t_ref, xs_ref, d_ref, w_ref, b_ref, z_ref):
    t = t_ref[0] + t_ref[1] + xs_ref[...]
    y = jnp.dot(t, w_ref[...], preferred_element_type=jnp.float32)
    z_ref[...] = jnp.maximum(y * d_ref[...] + b_ref[...], 0.0)


def _make_dense(interpret=False):
    blk = 1600
    return pl.pallas_call(
        _dense_body,
        grid=(NT // blk,),
        in_specs=[
            pl.BlockSpec((NC, blk, IN_DIM), lambda i: (0, i, 0)),
            pl.BlockSpec((blk, IN_DIM), lambda i: (i, 0)),
            pl.BlockSpec((blk, 1), lambda i: (i, 0)),
            pl.BlockSpec((IN_DIM, OUT_DIM), lambda i: (0, 0)),
            pl.BlockSpec((1, OUT_DIM), lambda i: (0, 0)),
        ],
        out_specs=pl.BlockSpec((blk, OUT_DIM), lambda i: (i, 0)),
        out_shape=jax.ShapeDtypeStruct((NT, OUT_DIM), jnp.float32),
        interpret=interpret,
    )


def _build(interpret=False):
    return (_make_deg(interpret), _make_scat(interpret), _make_dec(interpret),
            _make_dense(interpret))


def kernel(x, edge_index, W, b):
    deg_call, scat_call, dec_call, dense_call = _build()

    src = edge_index[0].astype(jnp.int32)
    dst = edge_index[1].astype(jnp.int32)
    pad = EP - E
    srcp = jnp.concatenate([src, jnp.zeros((pad,), jnp.int32)]).reshape(NW, K, B)
    dstp_s = jnp.concatenate([dst, jnp.full((pad,), N, jnp.int32)]).reshape(NW, K, B)
    dstp_d = jnp.concatenate([dst, jnp.zeros((pad,), jnp.int32)]).reshape(NW, K, B)

    degt = deg_call(dstp_s)
    deg = degt[0, :, 0] + degt[1, :, 0] + 1.0
    dinv = lax.rsqrt(deg)
    x_pad = jnp.concatenate([x, jnp.zeros((NT - N, IN_DIM), jnp.float32)])
    xs = x_pad * dinv[:, None]

    t_tab = scat_call(srcp, dstp_s, xs)
    z = dense_call(t_tab, xs, dinv.reshape(NT, 1), W, b.reshape(1, OUT_DIM))

    outr = dec_call(srcp, dstp_d, z)
    adj_pred = outr.reshape(EP)[:E]
    return (adj_pred, edge_index)


# speedup vs baseline: 3.9203x; 1.1629x over previous
import functools

import jax
import jax.numpy as jnp
from jax import lax
from jax.experimental import pallas as pl
from jax.experimental.pallas import tpu as pltpu
from jax.experimental.pallas import tpu_sc as plsc

N = 50000
NT = 51200
E = 800000
NC = 2
NS = 16
NW = NC * NS
B = 128
K = 196
EP = NW * K * B
CHK = 14
RPT = NT // NS
DEGW = 16
IN_DIM = 32
OUT_DIM = 128

_MESH = plsc.VectorSubcoreMesh(core_axis_name="c", subcore_axis_name="s")
_SC_PARAMS = pltpu.CompilerParams(use_tc_tiling_on_sc=False,
                                  needs_layout_passes=False)


def _zero_vmem(ref, nrow, ncol):
    z16 = jnp.zeros((16,), jnp.float32)

    def body(i, carry):
        for j in range(ncol // 16):
            ref[i, pl.ds(j * 16, 16)] = z16
        return carry

    lax.fori_loop(0, nrow, body, 0)


def _zero_table(tab, zer, s):
    def body(r, carry):
        pltpu.sync_copy(zer, tab.at[pl.ds(s * RPT + r * B, B)])
        return carry

    lax.fori_loop(0, RPT // B, body, 0)


def _deg_body(dst_hbm, out_hbm, tab, idx_v, val_v, zer_v):
    c = lax.axis_index("c")
    s = lax.axis_index("s")
    wid = c * NS + s
    one16 = jnp.ones((16,), jnp.float32)

    def fill(i, carry):
        val_v[i, :] = one16
        zer_v[i, :] = jnp.zeros((16,), jnp.float32)
        return carry

    lax.fori_loop(0, B, fill, 0)
    _zero_table(tab, zer_v, s)
    plsc.subcore_barrier()

    pltpu.sync_copy(dst_hbm.at[wid], idx_v)

    def body(g, carry):
        pltpu.sync_copy(val_v, tab.at[idx_v.at[g]], add=True)
        return carry

    lax.fori_loop(0, K, body, 0)
    plsc.subcore_barrier()
    pltpu.sync_copy(tab.at[pl.ds(s * RPT, RPT)], out_hbm.at[c, pl.ds(s * RPT, RPT)])


def _make_deg(interpret=False):
    return pl.kernel(
        _deg_body,
        out_type=jax.ShapeDtypeStruct((NC, NT, DEGW), jnp.float32),
        mesh=_MESH,
        scratch_types=[
            pltpu.VMEM_SHARED((NT, DEGW), jnp.float32),
            pltpu.VMEM((K, B), jnp.int32),
            pltpu.VMEM((B, DEGW), jnp.float32),
            pltpu.VMEM((B, DEGW), jnp.float32),
        ],
        interpret=interpret,
        compiler_params=_SC_PARAMS,
    )


def _scat_body(src_hbm, dst_hbm, xs_hbm, out_hbm, tab, sidx, didx,
               rowsa, rowsb, zer, ga, gb, sa, sb):
    c = lax.axis_index("c")
    s = lax.axis_index("s")
    wid = c * NS + s

    _zero_vmem(zer, B, IN_DIM)
    _zero_table(tab, zer, s)
    plsc.subcore_barrier()

    def issue_g(g, buf, sem):
        gg = jnp.minimum(g, CHK - 1)
        pltpu.async_copy(xs_hbm.at[sidx.at[gg]], buf, sem)

    def wait_g(buf, sem):
        pltpu.make_async_copy(xs_hbm.at[sidx.at[0]], buf, sem).wait()

    def issue_s(g, buf, sem):
        pltpu.async_copy(buf, tab.at[didx.at[g]], sem, add=True)

    def wait_s(g, buf, sem):
        pltpu.make_async_copy(buf, tab.at[didx.at[g]], sem).wait()

    def outer(o, carry):
        pltpu.sync_copy(src_hbm.at[wid, pl.ds(o * CHK, CHK)], sidx)
        pltpu.sync_copy(dst_hbm.at[wid, pl.ds(o * CHK, CHK)], didx)
        issue_g(0, rowsa, ga)
        issue_g(1, rowsb, gb)

        def body(i, carry2):
            g0 = 2 * i
            wait_g(rowsa, ga)
            issue_s(g0, rowsa, sa)
            wait_g(rowsb, gb)
            issue_s(g0 + 1, rowsb, sb)
            wait_s(g0, rowsa, sa)
            issue_g(g0 + 2, rowsa, ga)
            wait_s(g0 + 1, rowsb, sb)
            issue_g(g0 + 3, rowsb, gb)
            return carry2

        lax.fori_loop(0, CHK // 2, body, 0)
        wait_g(rowsa, ga)
        wait_g(rowsb, gb)
        return carry

    lax.fori_loop(0, K // CHK, outer, 0)
    plsc.subcore_barrier()
    pltpu.sync_copy(tab.at[pl.ds(s * RPT, RPT)], out_hbm.at[c, pl.ds(s * RPT, RPT)])


def _make_scat(interpret=False):
    return pl.kernel(
        _scat_body,
        out_type=jax.ShapeDtypeStruct((NC, NT, IN_DIM), jnp.float32),
        mesh=_MESH,
        scratch_types=[
            pltpu.VMEM_SHARED((NT, IN_DIM), jnp.float32),
            pltpu.VMEM((CHK, B), jnp.int32),
            pltpu.VMEM((CHK, B), jnp.int32),
            pltpu.VMEM((B, IN_DIM), jnp.float32),
            pltpu.VMEM((B, IN_DIM), jnp.float32),
            pltpu.VMEM((B, IN_DIM), jnp.float32),
            pltpu.SemaphoreType.DMA,
            pltpu.SemaphoreType.DMA,
            pltpu.SemaphoreType.DMA,
            pltpu.SemaphoreType.DMA,
        ],
        interpret=interpret,
        compiler_params=_SC_PARAMS,
    )


def _dec_body(src_hbm, dst_hbm, z_hbm, out_hbm, sidx, didx,
              zs0, zd0, zs1, zd1, res, stage, sa0, sb0, sa1, sb1):
    c = lax.axis_index("c")
    s = lax.axis_index("s")
    wid = c * NS + s
    lane = jnp.arange(16, dtype=jnp.int32)

    pltpu.sync_copy(src_hbm.at[wid], sidx)
    pltpu.sync_copy(dst_hbm.at[wid], didx)

    def issue(g, zsb, zdb, sa, sb):
        gg = jnp.minimum(g, K - 1)
        pltpu.async_copy(z_hbm.at[sidx.at[gg]], zsb, sa)
        pltpu.async_copy(z_hbm.at[didx.at[gg]], zdb, sb)

    def wait(zsb, zdb, sa, sb):
        pltpu.make_async_copy(z_hbm.at[sidx.at[0]], zsb, sa).wait()
        pltpu.make_async_copy(z_hbm.at[didx.at[0]], zdb, sb).wait()

    def compute(zsb, zdb, g):
        def grp(t, carry2):
            nv = OUT_DIM // 16

            def loads(k):
                e = t * 16 + k
                return ([zsb[e, pl.ds(j * 16, 16)] for j in range(nv)],
                        [zdb[e, pl.ds(j * 16, 16)] for j in range(nv)])

            cur = loads(0)
            for k in range(16):
                nxt = loads(k + 1) if k < 15 else None
                a, bb = cur
                prods = [a[j] * bb[j] for j in range(nv)]
                while len(prods) > 1:
                    prods = [prods[i] + prods[i + 1]
                             for i in range(0, len(prods), 2)]
                stage[k, :] = prods[0]
                cur = nxt
            sums = [jnp.zeros((16,), jnp.float32) for _ in range(4)]
            for j in range(16):
                col = (lane + j) & 15
                sums[j % 4] = sums[j % 4] + plsc.load_gather(stage, [lane, col])
            acc = (sums[0] + sums[1]) + (sums[2] + sums[3])
            res[pl.ds(t * 16, 16)] = 1.0 / (1.0 + jnp.exp(-acc)) + 1e-15
            return carry2

        lax.fori_loop(0, B // 16, grp, 0)
        pltpu.sync_copy(res, out_hbm.at[wid, g])

    issue(0, zs0, zd0, sa0, sb0)

    def outer(o, carry):
        g0 = 2 * o
        issue(g0 + 1, zs1, zd1, sa1, sb1)
        wait(zs0, zd0, sa0, sb0)
        compute(zs0, zd0, g0)
        issue(g0 + 2, zs0, zd0, sa0, sb0)
        wait(zs1, zd1, sa1, sb1)
        compute(zs1, zd1, g0 + 1)
        return carry

    lax.fori_loop(0, K // 2, outer, 0)
    wait(zs0, zd0, sa0, sb0)


def _make_dec(interpret=False):
    return pl.kernel(
        _dec_body,
        out_type=jax.ShapeDtypeStruct((NW, K, B), jnp.float32),
        mesh=_MESH,
        scratch_types=[
            pltpu.VMEM((K, B), jnp.int32),
            pltpu.VMEM((K, B), jnp.int32),
            pltpu.VMEM((B, OUT_DIM), jnp.float32),
            pltpu.VMEM((B, OUT_DIM), jnp.float32),
            pltpu.VMEM((B, OUT_DIM), jnp.float32),
            pltpu.VMEM((B, OUT_DIM), jnp.float32),
            pltpu.VMEM((B,), jnp.float32),
            pltpu.VMEM((16, 16), jnp.float32),
            pltpu.SemaphoreType.DMA,
            pltpu.SemaphoreType.DMA,
            pltpu.SemaphoreType.DMA,
            pltpu.SemaphoreType.DMA,
        ],
        interpret=interpret,
        compiler_params=_SC_PARAMS,
    )


def _dense_body(t_ref, xs_ref, d_ref, w_ref, b_ref, z_ref):
    t = t_ref[0] + t_ref[1] + xs_ref[...]
    y = jnp.dot(t, w_ref[...], preferred_element_type=jnp.float32)
    z_ref[...] = jnp.maximum(y * d_ref[...] + b_ref[...], 0.0)


def _make_dense(interpret=False):
    blk = 1600
    return pl.pallas_call(
        _dense_body,
        grid=(NT // blk,),
        in_specs=[
            pl.BlockSpec((NC, blk, IN_DIM), lambda i: (0, i, 0)),
            pl.BlockSpec((blk, IN_DIM), lambda i: (i, 0)),
            pl.BlockSpec((blk, 1), lambda i: (i, 0)),
            pl.BlockSpec((IN_DIM, OUT_DIM), lambda i: (0, 0)),
            pl.BlockSpec((1, OUT_DIM), lambda i: (0, 0)),
        ],
        out_specs=pl.BlockSpec((blk, OUT_DIM), lambda i: (i, 0)),
        out_shape=jax.ShapeDtypeStruct((NT, OUT_DIM), jnp.float32),
        interpret=interpret,
    )


def _build(interpret=False):
    return (_make_deg(interpret), _make_scat(interpret), _make_dec(interpret),
            _make_dense(interpret))


def kernel(x, edge_index, W, b):
    deg_call, scat_call, dec_call, dense_call = _build()

    src = edge_index[0].astype(jnp.int32)
    dst = edge_index[1].astype(jnp.int32)
    pad = EP - E
    srcp = jnp.concatenate([src, jnp.zeros((pad,), jnp.int32)]).reshape(NW, K, B)
    dstp_s = jnp.concatenate([dst, jnp.full((pad,), N, jnp.int32)]).reshape(NW, K, B)
    dstp_d = jnp.concatenate([dst, jnp.zeros((pad,), jnp.int32)]).reshape(NW, K, B)

    degt = deg_call(dstp_s)
    deg = degt[0, :, 0] + degt[1, :, 0] + 1.0
    dinv = lax.rsqrt(deg)
    x_pad = jnp.concatenate([x, jnp.zeros((NT - N, IN_DIM), jnp.float32)])
    xs = x_pad * dinv[:, None]

    t_tab = scat_call(srcp, dstp_s, xs)
    z = dense_call(t_tab, xs, dinv.reshape(NT, 1), W, b.reshape(1, OUT_DIM))

    outr = dec_call(srcp, dstp_d, z)
    adj_pred = outr.reshape(EP)[:E]
    return (adj_pred, edge_index)


# speedup vs baseline: 4.7206x; 1.2042x over previous
import functools

import jax
import jax.numpy as jnp
from jax import lax
from jax.experimental import pallas as pl
from jax.experimental.pallas import tpu as pltpu
from jax.experimental.pallas import tpu_sc as plsc

N = 50000
NT = 51200
E = 800000
NC = 2
NS = 16
NW = NC * NS
B = 128
K = 196
EP = NW * K * B
CHK = 14
RPT = NT // NS
DEGW = 16
IN_DIM = 32
OUT_DIM = 128

_MESH = plsc.VectorSubcoreMesh(core_axis_name="c", subcore_axis_name="s")
_SC_PARAMS = pltpu.CompilerParams(use_tc_tiling_on_sc=False,
                                  needs_layout_passes=False)


def _zero_vmem(ref, nrow, ncol):
    z16 = jnp.zeros((16,), jnp.float32)

    def body(i, carry):
        for j in range(ncol // 16):
            ref[i, pl.ds(j * 16, 16)] = z16
        return carry

    lax.fori_loop(0, nrow, body, 0)


def _zero_table(tab, zer, s):
    def body(r, carry):
        pltpu.sync_copy(zer, tab.at[pl.ds(s * RPT + r * B, B)])
        return carry

    lax.fori_loop(0, RPT // B, body, 0)


def _deg_body(dst_hbm, out_hbm, tab, idx_v, val_v, zer_v):
    c = lax.axis_index("c")
    s = lax.axis_index("s")
    wid = c * NS + s
    one16 = jnp.ones((16,), jnp.float32)

    def fill(i, carry):
        val_v[i, :] = one16
        zer_v[i, :] = jnp.zeros((16,), jnp.float32)
        return carry

    lax.fori_loop(0, B, fill, 0)
    _zero_table(tab, zer_v, s)
    plsc.subcore_barrier()

    pltpu.sync_copy(dst_hbm.at[wid], idx_v)

    def body(g, carry):
        pltpu.sync_copy(val_v, tab.at[idx_v.at[g]], add=True)
        return carry

    lax.fori_loop(0, K, body, 0)
    plsc.subcore_barrier()
    pltpu.sync_copy(tab.at[pl.ds(s * RPT, RPT)], out_hbm.at[c, pl.ds(s * RPT, RPT)])


def _make_deg(interpret=False):
    return pl.kernel(
        _deg_body,
        out_type=jax.ShapeDtypeStruct((NC, NT, DEGW), jnp.float32),
        mesh=_MESH,
        scratch_types=[
            pltpu.VMEM_SHARED((NT, DEGW), jnp.float32),
            pltpu.VMEM((K, B), jnp.int32),
            pltpu.VMEM((B, DEGW), jnp.float32),
            pltpu.VMEM((B, DEGW), jnp.float32),
        ],
        interpret=interpret,
        compiler_params=_SC_PARAMS,
    )


def _scat_body(src_hbm, dst_hbm, xs_hbm, out_hbm, tab, sidx, didx,
               rowsa, rowsb, zer, ga, gb, sa, sb):
    c = lax.axis_index("c")
    s = lax.axis_index("s")
    wid = c * NS + s

    _zero_vmem(zer, B, IN_DIM)
    _zero_table(tab, zer, s)
    plsc.subcore_barrier()

    def issue_g(g, buf, sem):
        gg = jnp.minimum(g, CHK - 1)
        pltpu.async_copy(xs_hbm.at[sidx.at[gg]], buf, sem)

    def wait_g(buf, sem):
        pltpu.make_async_copy(xs_hbm.at[sidx.at[0]], buf, sem).wait()

    def issue_s(g, buf, sem):
        pltpu.async_copy(buf, tab.at[didx.at[g]], sem, add=True)

    def wait_s(g, buf, sem):
        pltpu.make_async_copy(buf, tab.at[didx.at[g]], sem).wait()

    def outer(o, carry):
        pltpu.sync_copy(src_hbm.at[wid, pl.ds(o * CHK, CHK)], sidx)
        pltpu.sync_copy(dst_hbm.at[wid, pl.ds(o * CHK, CHK)], didx)
        issue_g(0, rowsa, ga)
        issue_g(1, rowsb, gb)

        def body(i, carry2):
            g0 = 2 * i
            wait_g(rowsa, ga)
            issue_s(g0, rowsa, sa)
            wait_g(rowsb, gb)
            issue_s(g0 + 1, rowsb, sb)
            wait_s(g0, rowsa, sa)
            issue_g(g0 + 2, rowsa, ga)
            wait_s(g0 + 1, rowsb, sb)
            issue_g(g0 + 3, rowsb, gb)
            return carry2

        lax.fori_loop(0, CHK // 2, body, 0)
        wait_g(rowsa, ga)
        wait_g(rowsb, gb)
        return carry

    lax.fori_loop(0, K // CHK, outer, 0)
    plsc.subcore_barrier()
    pltpu.sync_copy(tab.at[pl.ds(s * RPT, RPT)], out_hbm.at[c, pl.ds(s * RPT, RPT)])


def _make_scat(interpret=False):
    return pl.kernel(
        _scat_body,
        out_type=jax.ShapeDtypeStruct((NC, NT, IN_DIM), jnp.float32),
        mesh=_MESH,
        scratch_types=[
            pltpu.VMEM_SHARED((NT, IN_DIM), jnp.float32),
            pltpu.VMEM((CHK, B), jnp.int32),
            pltpu.VMEM((CHK, B), jnp.int32),
            pltpu.VMEM((B, IN_DIM), jnp.float32),
            pltpu.VMEM((B, IN_DIM), jnp.float32),
            pltpu.VMEM((B, IN_DIM), jnp.float32),
            pltpu.SemaphoreType.DMA,
            pltpu.SemaphoreType.DMA,
            pltpu.SemaphoreType.DMA,
            pltpu.SemaphoreType.DMA,
        ],
        interpret=interpret,
        compiler_params=_SC_PARAMS,
    )


def _dec_body(src_hbm, dst_hbm, z_hbm, out_hbm, sidx, didx,
              zs0, zd0, zs1, zd1, res, stage, sa0, sb0, sa1, sb1):
    c = lax.axis_index("c")
    s = lax.axis_index("s")
    wid = c * NS + s
    lane = jnp.arange(16, dtype=jnp.int32)

    pltpu.sync_copy(src_hbm.at[wid], sidx)
    pltpu.sync_copy(dst_hbm.at[wid], didx)

    def issue(g, zsb, zdb, sa, sb):
        gg = jnp.minimum(g, K - 1)
        pltpu.async_copy(z_hbm.at[sidx.at[gg]], zsb, sa)
        pltpu.async_copy(z_hbm.at[didx.at[gg]], zdb, sb)

    def wait(zsb, zdb, sa, sb):
        pltpu.make_async_copy(z_hbm.at[sidx.at[0]], zsb, sa).wait()
        pltpu.make_async_copy(z_hbm.at[didx.at[0]], zdb, sb).wait()

    def compute(zsb, zdb, g):
        def grp(t, carry2):
            nv = OUT_DIM // 16

            def loads(k):
                e = t * 16 + k
                aa, bb2 = [], []
                for j in range(OUT_DIM // 32):
                    a0, a1 = plsc.unpack(zsb[e, pl.ds(j * 32, 32)],
                                         format=plsc.PackFormat.INTERLEAVED)
                    b0, b1 = plsc.unpack(zdb[e, pl.ds(j * 32, 32)],
                                         format=plsc.PackFormat.INTERLEAVED)
                    aa += [a0, a1]
                    bb2 += [b0, b1]
                return (aa, bb2)

            cur = loads(0)
            for k in range(16):
                nxt = loads(k + 1) if k < 15 else None
                a, bb = cur
                prods = [a[j] * bb[j] for j in range(nv)]
                while len(prods) > 1:
                    prods = [prods[i] + prods[i + 1]
                             for i in range(0, len(prods), 2)]
                stage[k, :] = prods[0]
                cur = nxt
            sums = [jnp.zeros((16,), jnp.float32) for _ in range(4)]
            for j in range(16):
                col = (lane + j) & 15
                sums[j % 4] = sums[j % 4] + plsc.load_gather(stage, [lane, col])
            acc = (sums[0] + sums[1]) + (sums[2] + sums[3])
            res[pl.ds(t * 16, 16)] = 1.0 / (1.0 + jnp.exp(-acc)) + 1e-15
            return carry2

        lax.fori_loop(0, B // 16, grp, 0)
        pltpu.sync_copy(res, out_hbm.at[wid, g])

    issue(0, zs0, zd0, sa0, sb0)

    def outer(o, carry):
        g0 = 2 * o
        issue(g0 + 1, zs1, zd1, sa1, sb1)
        wait(zs0, zd0, sa0, sb0)
        compute(zs0, zd0, g0)
        issue(g0 + 2, zs0, zd0, sa0, sb0)
        wait(zs1, zd1, sa1, sb1)
        compute(zs1, zd1, g0 + 1)
        return carry

    lax.fori_loop(0, K // 2, outer, 0)
    wait(zs0, zd0, sa0, sb0)


def _make_dec(interpret=False):
    return pl.kernel(
        _dec_body,
        out_type=jax.ShapeDtypeStruct((NW, K, B), jnp.float32),
        mesh=_MESH,
        scratch_types=[
            pltpu.VMEM((K, B), jnp.int32),
            pltpu.VMEM((K, B), jnp.int32),
            pltpu.VMEM((B, OUT_DIM), jnp.bfloat16),
            pltpu.VMEM((B, OUT_DIM), jnp.bfloat16),
            pltpu.VMEM((B, OUT_DIM), jnp.bfloat16),
            pltpu.VMEM((B, OUT_DIM), jnp.bfloat16),
            pltpu.VMEM((B,), jnp.float32),
            pltpu.VMEM((16, 16), jnp.float32),
            pltpu.SemaphoreType.DMA,
            pltpu.SemaphoreType.DMA,
            pltpu.SemaphoreType.DMA,
            pltpu.SemaphoreType.DMA,
        ],
        interpret=interpret,
        compiler_params=_SC_PARAMS,
    )


def _dense_body(t_ref, xs_ref, d_ref, w_ref, b_ref, z_ref):
    t = t_ref[0] + t_ref[1] + xs_ref[...]
    y = jnp.dot(t, w_ref[...], preferred_element_type=jnp.float32)
    z = jnp.maximum(y * d_ref[...] + b_ref[...], 0.0)
    z_ref[...] = z.astype(jnp.bfloat16)


def _make_dense(interpret=False):
    blk = 1600
    return pl.pallas_call(
        _dense_body,
        grid=(NT // blk,),
        in_specs=[
            pl.BlockSpec((NC, blk, IN_DIM), lambda i: (0, i, 0)),
            pl.BlockSpec((blk, IN_DIM), lambda i: (i, 0)),
            pl.BlockSpec((blk, 1), lambda i: (i, 0)),
            pl.BlockSpec((IN_DIM, OUT_DIM), lambda i: (0, 0)),
            pl.BlockSpec((1, OUT_DIM), lambda i: (0, 0)),
        ],
        out_specs=pl.BlockSpec((blk, OUT_DIM), lambda i: (i, 0)),
        out_shape=jax.ShapeDtypeStruct((NT, OUT_DIM), jnp.bfloat16),
        interpret=interpret,
    )


def _build(interpret=False):
    return (_make_deg(interpret), _make_scat(interpret), _make_dec(interpret),
            _make_dense(interpret))


def kernel(x, edge_index, W, b):
    deg_call, scat_call, dec_call, dense_call = _build()

    src = edge_index[0].astype(jnp.int32)
    dst = edge_index[1].astype(jnp.int32)
    pad = EP - E
    srcp = jnp.concatenate([src, jnp.zeros((pad,), jnp.int32)]).reshape(NW, K, B)
    dstp_s = jnp.concatenate([dst, jnp.full((pad,), N, jnp.int32)]).reshape(NW, K, B)
    dstp_d = jnp.concatenate([dst, jnp.zeros((pad,), jnp.int32)]).reshape(NW, K, B)

    degt = deg_call(dstp_s)
    deg = degt[0, :, 0] + degt[1, :, 0] + 1.0
    dinv = lax.rsqrt(deg)
    x_pad = jnp.concatenate([x, jnp.zeros((NT - N, IN_DIM), jnp.float32)])
    xs = x_pad * dinv[:, None]

    t_tab = scat_call(srcp, dstp_s, xs)
    z = dense_call(t_tab, xs, dinv.reshape(NT, 1), W, b.reshape(1, OUT_DIM))

    outr = dec_call(srcp, dstp_d, z)
    adj_pred = outr.reshape(EP)[:E]
    return (adj_pred, edge_index)


# speedup vs baseline: 5.0143x; 1.0622x over previous
import functools

import jax
import jax.numpy as jnp
from jax import lax
from jax.experimental import pallas as pl
from jax.experimental.pallas import tpu as pltpu
from jax.experimental.pallas import tpu_sc as plsc

N = 50000
NT = 51200
E = 800000
NC = 2
NS = 16
NW = NC * NS
B = 128
K = 196
EP = NW * K * B
CHK = 28
RPT = NT // NS
DEGW = 16
IN_DIM = 32
OUT_DIM = 128

_MESH = plsc.VectorSubcoreMesh(core_axis_name="c", subcore_axis_name="s")
_SC_PARAMS = pltpu.CompilerParams(use_tc_tiling_on_sc=False,
                                  needs_layout_passes=False)


def _zero_vmem(ref, nrow, ncol):
    z16 = jnp.zeros((16,), jnp.float32)

    def body(i, carry):
        for j in range(ncol // 16):
            ref[i, pl.ds(j * 16, 16)] = z16
        return carry

    lax.fori_loop(0, nrow, body, 0)


def _zero_table(tab, zer, s):
    def body(r, carry):
        pltpu.sync_copy(zer, tab.at[pl.ds(s * RPT + r * B, B)])
        return carry

    lax.fori_loop(0, RPT // B, body, 0)


def _deg_body(dst_hbm, out_hbm, tab, idx_v, val_v, zer_v):
    c = lax.axis_index("c")
    s = lax.axis_index("s")
    wid = c * NS + s
    one16 = jnp.ones((16,), jnp.float32)

    def fill(i, carry):
        val_v[i, :] = one16
        zer_v[i, :] = jnp.zeros((16,), jnp.float32)
        return carry

    lax.fori_loop(0, B, fill, 0)
    _zero_table(tab, zer_v, s)
    plsc.subcore_barrier()

    pltpu.sync_copy(dst_hbm.at[wid], idx_v)

    def body(g, carry):
        pltpu.sync_copy(val_v, tab.at[idx_v.at[g]], add=True)
        return carry

    lax.fori_loop(0, K, body, 0)
    plsc.subcore_barrier()
    pltpu.sync_copy(tab.at[pl.ds(s * RPT, RPT)], out_hbm.at[c, pl.ds(s * RPT, RPT)])


def _make_deg(interpret=False):
    return pl.kernel(
        _deg_body,
        out_type=jax.ShapeDtypeStruct((NC, NT, DEGW), jnp.float32),
        mesh=_MESH,
        scratch_types=[
            pltpu.VMEM_SHARED((NT, DEGW), jnp.float32),
            pltpu.VMEM((K, B), jnp.int32),
            pltpu.VMEM((B, DEGW), jnp.float32),
            pltpu.VMEM((B, DEGW), jnp.float32),
        ],
        interpret=interpret,
        compiler_params=_SC_PARAMS,
    )


def _scat_body(src_hbm, dst_hbm, xs_hbm, out_hbm, tab, sidx, didx,
               r0, r1, r2, r3, g0s, g1s, g2s, g3s, s0s, s1s, s2s, s3s):
    c = lax.axis_index("c")
    s = lax.axis_index("s")
    wid = c * NS + s
    bufs = ((r0, g0s, s0s), (r1, g1s, s1s), (r2, g2s, s2s), (r3, g3s, s3s))

    _zero_vmem(r0, B, IN_DIM)
    _zero_table(tab, r0, s)
    plsc.subcore_barrier()

    def issue_g(g, buf, sem):
        gg = jnp.minimum(g, CHK - 1)
        pltpu.async_copy(xs_hbm.at[sidx.at[gg]], buf, sem)

    def wait_g(buf, sem):
        pltpu.make_async_copy(xs_hbm.at[sidx.at[0]], buf, sem).wait()

    def issue_s(g, buf, sem):
        pltpu.async_copy(buf, tab.at[didx.at[g]], sem, add=True)

    def wait_s(g, buf, sem):
        pltpu.make_async_copy(buf, tab.at[didx.at[g]], sem).wait()

    def outer(o, carry):
        pltpu.sync_copy(src_hbm.at[wid, pl.ds(o * CHK, CHK)], sidx)
        pltpu.sync_copy(dst_hbm.at[wid, pl.ds(o * CHK, CHK)], didx)
        for b, (buf, gsem, _) in enumerate(bufs):
            issue_g(b, buf, gsem)

        def body(i, carry2):
            base = 4 * i
            for b, (buf, gsem, ssem) in enumerate(bufs):
                wait_g(buf, gsem)
                issue_s(base + b, buf, ssem)
            for b, (buf, gsem, ssem) in enumerate(bufs):
                wait_s(base + b, buf, ssem)
                issue_g(base + b + 4, buf, gsem)
            return carry2

        lax.fori_loop(0, CHK // 4, body, 0)
        for buf, gsem, _ in bufs:
            wait_g(buf, gsem)
        return carry

    lax.fori_loop(0, K // CHK, outer, 0)
    plsc.subcore_barrier()
    pltpu.sync_copy(tab.at[pl.ds(s * RPT, RPT)], out_hbm.at[c, pl.ds(s * RPT, RPT)])


def _make_scat(interpret=False):
    return pl.kernel(
        _scat_body,
        out_type=jax.ShapeDtypeStruct((NC, NT, IN_DIM), jnp.float32),
        mesh=_MESH,
        scratch_types=(
            [pltpu.VMEM_SHARED((NT, IN_DIM), jnp.float32),
             pltpu.VMEM((CHK, B), jnp.int32),
             pltpu.VMEM((CHK, B), jnp.int32)]
            + [pltpu.VMEM((B, IN_DIM), jnp.float32)] * 4
            + [pltpu.SemaphoreType.DMA] * 8
        ),
        interpret=interpret,
        compiler_params=_SC_PARAMS,
    )


def _dec_body(src_hbm, dst_hbm, z_hbm, out_hbm, sidx, didx,
              zs0, zd0, zs1, zd1, res, stage, sa0, sb0, sa1, sb1):
    c = lax.axis_index("c")
    s = lax.axis_index("s")
    wid = c * NS + s
    lane = jnp.arange(16, dtype=jnp.int32)

    pltpu.sync_copy(src_hbm.at[wid], sidx)
    pltpu.sync_copy(dst_hbm.at[wid], didx)

    def issue(g, zsb, zdb, sa, sb):
        gg = jnp.minimum(g, K - 1)
        pltpu.async_copy(z_hbm.at[sidx.at[gg]], zsb, sa)
        pltpu.async_copy(z_hbm.at[didx.at[gg]], zdb, sb)

    def wait(zsb, zdb, sa, sb):
        pltpu.make_async_copy(z_hbm.at[sidx.at[0]], zsb, sa).wait()
        pltpu.make_async_copy(z_hbm.at[didx.at[0]], zdb, sb).wait()

    def compute(zsb, zdb, g):
        def grp(t, carry2):
            nv = OUT_DIM // 16

            def loads(k):
                e = t * 16 + k
                aa, bb2 = [], []
                for j in range(OUT_DIM // 32):
                    a0, a1 = plsc.unpack(zsb[e, pl.ds(j * 32, 32)],
                                         format=plsc.PackFormat.INTERLEAVED)
                    b0, b1 = plsc.unpack(zdb[e, pl.ds(j * 32, 32)],
                                         format=plsc.PackFormat.INTERLEAVED)
                    aa += [a0, a1]
                    bb2 += [b0, b1]
                return (aa, bb2)

            cur = loads(0)
            for k in range(16):
                nxt = loads(k + 1) if k < 15 else None
                a, bb = cur
                prods = [a[j] * bb[j] for j in range(nv)]
                while len(prods) > 1:
                    prods = [prods[i] + prods[i + 1]
                             for i in range(0, len(prods), 2)]
                stage[k, :] = prods[0]
                cur = nxt
            sums = [jnp.zeros((16,), jnp.float32) for _ in range(4)]
            for j in range(16):
                col = (lane + j) & 15
                sums[j % 4] = sums[j % 4] + plsc.load_gather(stage, [lane, col])
            acc = (sums[0] + sums[1]) + (sums[2] + sums[3])
            res[pl.ds(t * 16, 16)] = 1.0 / (1.0 + jnp.exp(-acc)) + 1e-15
            return carry2

        lax.fori_loop(0, B // 16, grp, 0)
        pltpu.sync_copy(res, out_hbm.at[wid, g])

    issue(0, zs0, zd0, sa0, sb0)

    def outer(o, carry):
        g0 = 2 * o
        issue(g0 + 1, zs1, zd1, sa1, sb1)
        wait(zs0, zd0, sa0, sb0)
        compute(zs0, zd0, g0)
        issue(g0 + 2, zs0, zd0, sa0, sb0)
        wait(zs1, zd1, sa1, sb1)
        compute(zs1, zd1, g0 + 1)
        return carry

    lax.fori_loop(0, K // 2, outer, 0)
    wait(zs0, zd0, sa0, sb0)


def _make_dec(interpret=False):
    return pl.kernel(
        _dec_body,
        out_type=jax.ShapeDtypeStruct((NW, K, B), jnp.float32),
        mesh=_MESH,
        scratch_types=[
            pltpu.VMEM((K, B), jnp.int32),
            pltpu.VMEM((K, B), jnp.int32),
            pltpu.VMEM((B, OUT_DIM), jnp.bfloat16),
            pltpu.VMEM((B, OUT_DIM), jnp.bfloat16),
            pltpu.VMEM((B, OUT_DIM), jnp.bfloat16),
            pltpu.VMEM((B, OUT_DIM), jnp.bfloat16),
            pltpu.VMEM((B,), jnp.float32),
            pltpu.VMEM((16, 16), jnp.float32),
            pltpu.SemaphoreType.DMA,
            pltpu.SemaphoreType.DMA,
            pltpu.SemaphoreType.DMA,
            pltpu.SemaphoreType.DMA,
        ],
        interpret=interpret,
        compiler_params=_SC_PARAMS,
    )


def _dense_body(t_ref, xs_ref, d_ref, w_ref, b_ref, z_ref):
    t = t_ref[0] + t_ref[1] + xs_ref[...]
    y = jnp.dot(t, w_ref[...], preferred_element_type=jnp.float32)
    z = jnp.maximum(y * d_ref[...] + b_ref[...], 0.0)
    z_ref[...] = z.astype(jnp.bfloat16)


def _make_dense(interpret=False):
    blk = 1600
    return pl.pallas_call(
        _dense_body,
        grid=(NT // blk,),
        in_specs=[
            pl.BlockSpec((NC, blk, IN_DIM), lambda i: (0, i, 0)),
            pl.BlockSpec((blk, IN_DIM), lambda i: (i, 0)),
            pl.BlockSpec((blk, 1), lambda i: (i, 0)),
            pl.BlockSpec((IN_DIM, OUT_DIM), lambda i: (0, 0)),
            pl.BlockSpec((1, OUT_DIM), lambda i: (0, 0)),
        ],
        out_specs=pl.BlockSpec((blk, OUT_DIM), lambda i: (i, 0)),
        out_shape=jax.ShapeDtypeStruct((NT, OUT_DIM), jnp.bfloat16),
        interpret=interpret,
    )


def _build(interpret=False):
    return (_make_deg(interpret), _make_scat(interpret), _make_dec(interpret),
            _make_dense(interpret))


def kernel(x, edge_index, W, b):
    deg_call, scat_call, dec_call, dense_call = _build()

    src = edge_index[0].astype(jnp.int32)
    dst = edge_index[1].astype(jnp.int32)
    pad = EP - E
    srcp = jnp.concatenate([src, jnp.zeros((pad,), jnp.int32)]).reshape(NW, K, B)
    dstp_s = jnp.concatenate([dst, jnp.full((pad,), N, jnp.int32)]).reshape(NW, K, B)
    dstp_d = jnp.concatenate([dst, jnp.zeros((pad,), jnp.int32)]).reshape(NW, K, B)

    degt = deg_call(dstp_s)
    deg = degt[0, :, 0] + degt[1, :, 0] + 1.0
    dinv = lax.rsqrt(deg)
    x_pad = jnp.concatenate([x, jnp.zeros((NT - N, IN_DIM), jnp.float32)])
    xs = x_pad * dinv[:, None]

    t_tab = scat_call(srcp, dstp_s, xs)
    z = dense_call(t_tab, xs, dinv.reshape(NT, 1), W, b.reshape(1, OUT_DIM))

    outr = dec_call(srcp, dstp_d, z)
    adj_pred = outr.reshape(EP)[:E]
    return (adj_pred, edge_index)
